# baseline (speedup 1.0000x reference)
import jax
import jax.numpy as jnp
from jax import lax
from jax.experimental import pallas as pl
from jax.experimental.pallas import tpu as pltpu
from jax.experimental.pallas import tpu_sc as plsc

N = 10000
E = 320000
D = 128
DE = 16
G = 64

NC = 2
NS = 16
NW = NC * NS
C = 128
NCHUNK = 79
CH0 = 60
CH1 = 2 * NCHUNK - CH0
EPW = NCHUNK * C
E_PAD = NW * EPW
GCHUNK = E_PAD // C
N_PAD = 10112
RPT = N_PAD // NS
ZROW = N

EB = 10112
ESTEPS = E_PAD // EB


def _prep_body(x_ref, wq_ref, wk_ref, wv_ref, wr_ref, we_ref, b_ref,
               eattr_ref, q_ref, k_ref, v_ref, xr_ref, ea_ref):
    @pl.when(pl.program_id(0) == 0)
    def _():
        xv = x_ref[...]
        q_ref[...] = jnp.dot(xv, wq_ref[...], preferred_element_type=jnp.float32)
        k_ref[...] = jnp.dot(xv, wk_ref[...], preferred_element_type=jnp.float32)
        v_ref[...] = jnp.dot(xv, wv_ref[...], preferred_element_type=jnp.float32)
        xr_ref[...] = (
            jnp.dot(xv, wr_ref[...], preferred_element_type=jnp.float32)
            + b_ref[...]
        )

    ea_ref[...] = jnp.dot(eattr_ref[...], we_ref[...],
                          preferred_element_type=jnp.float32)


_prep = pl.pallas_call(
    _prep_body,
    grid=(ESTEPS,),
    in_specs=[
        pl.BlockSpec((N_PAD, D), lambda i: (0, 0)),
        pl.BlockSpec((D, D), lambda i: (0, 0)),
        pl.BlockSpec((D, D), lambda i: (0, 0)),
        pl.BlockSpec((D, D), lambda i: (0, 0)),
        pl.BlockSpec((D, D), lambda i: (0, 0)),
        pl.BlockSpec((DE, D), lambda i: (0, 0)),
        pl.BlockSpec((1, D), lambda i: (0, 0)),
        pl.BlockSpec((EB, DE), lambda i: (i, 0)),
    ],
    out_specs=[
        pl.BlockSpec((N_PAD, D), lambda i: (0, 0)),
        pl.BlockSpec((N_PAD, D), lambda i: (0, 0)),
        pl.BlockSpec((N_PAD, D), lambda i: (0, 0)),
        pl.BlockSpec((N_PAD, D), lambda i: (0, 0)),
        pl.BlockSpec((EB, D), lambda i: (i, 0)),
    ],
    out_shape=[
        jax.ShapeDtypeStruct((N_PAD, D), jnp.float32),
        jax.ShapeDtypeStruct((N_PAD, D), jnp.float32),
        jax.ShapeDtypeStruct((N_PAD, D), jnp.float32),
        jax.ShapeDtypeStruct((N_PAD, D), jnp.float32),
        jax.ShapeDtypeStruct((E_PAD, D), jnp.float32),
    ],
)


def _edge_body(q_hbm, k_hbm, v_hbm, ea_hbm, dst_hbm, src_hbm, out_hbm,
               za, vb, dsti, srci, agg_sh, sea, sq, sk, sv, ssc):
    cid = lax.axis_index("c")
    sid = lax.axis_index("s")
    nch = CH0 + cid * (CH1 - CH0)
    base = cid * NS * CH0 + sid * nch

    def zrow(r, carry):
        for j in range(D // 16):
            za[0, r, pl.ds(j * 16, 16)] = jnp.zeros((16,), jnp.float32)
        return carry

    lax.fori_loop(0, C, zrow, 0)
    for t in range(4):
        pltpu.sync_copy(za.at[0], agg_sh.at[pl.ds(sid * RPT + t * C, C)])
    pltpu.sync_copy(za.at[0, pl.ds(0, RPT - 4 * C)],
                    agg_sh.at[pl.ds(sid * RPT + 4 * C, RPT - 4 * C)])
    plsc.subcore_barrier()

    def ea_slice(c):
        return ea_hbm.at[pl.ds((base + c) * C, C)]

    pltpu.sync_copy(dst_hbm.at[base], dsti.at[0])
    pltpu.sync_copy(src_hbm.at[base], srci.at[0])
    pltpu.sync_copy(ea_slice(0), za.at[0])
    pltpu.async_copy(q_hbm.at[dsti.at[0]], za.at[0], sq, add=True).wait()
    pltpu.async_copy(k_hbm.at[srci.at[0]], za.at[0], sk, add=True)
    pltpu.async_copy(v_hbm.at[srci.at[0]], vb, sv)

    def chunk_step(c, par):
        op = 1 - par
        @pl.when(c >= 1)
        def _():
            pltpu.make_async_copy(za.at[op], agg_sh.at[dsti.at[op]],
                                  ssc).wait()

        has_next = c <= nch - 2

        @pl.when(has_next)
        def _():
            pltpu.sync_copy(dst_hbm.at[base + c + 1], dsti.at[op])
            pltpu.sync_copy(src_hbm.at[base + c + 1], srci.at[op])
            pltpu.async_copy(ea_slice(c + 1), za.at[op], sea)

        pltpu.make_async_copy(k_hbm.at[srci.at[par]], za.at[par], sk).wait()
        pltpu.make_async_copy(v_hbm.at[srci.at[par]], vb, sv).wait()

        def row(r, rcarry):
            for j in range(D // 16):
                sl = pl.ds(j * 16, 16)
                za[par, r, sl] = vb[r, sl] / (1.0 + jnp.exp(-za[par, r, sl]))
            return rcarry

        lax.fori_loop(0, C, row, 0)
        pltpu.async_copy(za.at[par], agg_sh.at[dsti.at[par]], ssc, add=True)

        @pl.when(has_next)
        def _():
            pltpu.make_async_copy(ea_slice(c + 1), za.at[op], sea).wait()
            pltpu.async_copy(q_hbm.at[dsti.at[op]], za.at[op], sq, add=True)
            pltpu.async_copy(v_hbm.at[srci.at[op]], vb, sv)
            pltpu.make_async_copy(q_hbm.at[dsti.at[op]], za.at[op], sq).wait()
            pltpu.async_copy(k_hbm.at[srci.at[op]], za.at[op], sk, add=True)

    def pair(t, carry):
        chunk_step(2 * t, 0)
        chunk_step(2 * t + 1, 1)
        return carry

    lax.fori_loop(0, nch // 2, pair, 0)
    pltpu.make_async_copy(za.at[1], agg_sh.at[dsti.at[1]], ssc).wait()
    plsc.subcore_barrier()

    for t in range(4):
        off = sid * RPT + t * C
        pltpu.sync_copy(agg_sh.at[pl.ds(off, C)], za.at[0])
        pltpu.sync_copy(za.at[0], out_hbm.at[cid, pl.ds(off, C)])
    tail = RPT - 4 * C
    off = sid * RPT + 4 * C
    pltpu.sync_copy(agg_sh.at[pl.ds(off, tail)], za.at[0, pl.ds(0, tail)])
    pltpu.sync_copy(za.at[0, pl.ds(0, tail)], out_hbm.at[cid, pl.ds(off, tail)])


_edge = pl.kernel(
    _edge_body,
    out_type=jax.ShapeDtypeStruct((NC, N_PAD, D), jnp.float32),
    mesh=plsc.VectorSubcoreMesh(core_axis_name="c", subcore_axis_name="s"),
    scratch_types=[
        pltpu.VMEM((2, C, D), jnp.float32),
        pltpu.VMEM((C, D), jnp.float32),
        pltpu.VMEM((2, C), jnp.int32),
        pltpu.VMEM((2, C), jnp.int32),
        pltpu.VMEM_SHARED((N_PAD, D), jnp.float32),
        pltpu.SemaphoreType.DMA,
        pltpu.SemaphoreType.DMA,
        pltpu.SemaphoreType.DMA,
        pltpu.SemaphoreType.DMA,
        pltpu.SemaphoreType.DMA,
    ],
)


def _pool_body(xr_ref, a0_ref, a1_ref, batch_ref, out_ref):
    out = jnp.maximum(xr_ref[...] + a0_ref[...] + a1_ref[...], 0.0)
    sel = (batch_ref[...] ==
           lax.broadcasted_iota(jnp.int32, (N, G), 1)).astype(jnp.float32)
    psum = lax.dot_general(sel, out, (((0,), (0,)), ((), ())),
                           preferred_element_type=jnp.float32)
    cnts = lax.dot_general(sel, jnp.ones((N, D), jnp.float32),
                           (((0,), (0,)), ((), ())),
                           preferred_element_type=jnp.float32)
    out_ref[...] = psum / jnp.maximum(cnts, 1.0)


_pool = pl.pallas_call(
    _pool_body,
    out_shape=jax.ShapeDtypeStruct((G, D), jnp.float32),
)


def kernel(x, edge_index, edge_attr, batch, Wq, Wk, Wv, We, Wroot, b):
    pad_e = E_PAD - E
    src3 = jnp.concatenate(
        [edge_index[0], jnp.full((pad_e,), ZROW, jnp.int32)]
    ).reshape(GCHUNK, C)
    dst3 = jnp.concatenate(
        [edge_index[1], jnp.zeros((pad_e,), jnp.int32)]
    ).reshape(GCHUNK, C)
    x_pad = jnp.pad(x, ((0, N_PAD - N), (0, 0)))
    ea_pad = jnp.pad(edge_attr, ((0, pad_e), (0, 0)))
    q, k, v, xr, ea = _prep(x_pad, Wq, Wk, Wv, Wroot, We, b.reshape(1, D),
                            ea_pad)
    aggs = _edge(q, k, v, ea, dst3, src3)
    return _pool(xr[:N], aggs[0, :N], aggs[1, :N], batch.reshape(N, 1))


# speedup vs baseline: 1.1726x; 1.1726x over previous
import jax
import jax.numpy as jnp
from jax import lax
from jax.experimental import pallas as pl
from jax.experimental.pallas import tpu as pltpu
from jax.experimental.pallas import tpu_sc as plsc

N = 10000
E = 320000
D = 128
DE = 16
G = 64

NC = 2
NS = 16
NW = NC * NS
C = 128
NCHUNK = 79
CH0 = 98
CH1 = 2 * NCHUNK - CH0
EPW = NCHUNK * C
E_PAD = NW * EPW
GCHUNK = E_PAD // C
N_PAD = 10112
RPT = N_PAD // NS
ZROW = N

EB = 10112
ESTEPS = E_PAD // EB


def _prep_body(x_ref, wq_ref, wk_ref, wv_ref, wr_ref, we_ref, b_ref,
               eattr_ref, q_ref, k_ref, v_ref, xr_ref, ea_ref):
    @pl.when(pl.program_id(0) == 0)
    def _():
        xv = x_ref[...]
        q_ref[...] = jnp.dot(xv, wq_ref[...], preferred_element_type=jnp.float32)
        k_ref[...] = jnp.dot(xv, wk_ref[...], preferred_element_type=jnp.float32)
        v_ref[...] = jnp.dot(xv, wv_ref[...], preferred_element_type=jnp.float32)
        xr_ref[...] = (
            jnp.dot(xv, wr_ref[...], preferred_element_type=jnp.float32)
            + b_ref[...]
        )

    ea_ref[...] = jnp.dot(eattr_ref[...], we_ref[...],
                          preferred_element_type=jnp.float32)


_prep = pl.pallas_call(
    _prep_body,
    grid=(ESTEPS,),
    in_specs=[
        pl.BlockSpec((N_PAD, D), lambda i: (0, 0)),
        pl.BlockSpec((D, D), lambda i: (0, 0)),
        pl.BlockSpec((D, D), lambda i: (0, 0)),
        pl.BlockSpec((D, D), lambda i: (0, 0)),
        pl.BlockSpec((D, D), lambda i: (0, 0)),
        pl.BlockSpec((DE, D), lambda i: (0, 0)),
        pl.BlockSpec((1, D), lambda i: (0, 0)),
        pl.BlockSpec((EB, DE), lambda i: (i, 0)),
    ],
    out_specs=[
        pl.BlockSpec((N_PAD, D), lambda i: (0, 0)),
        pl.BlockSpec((N_PAD, D), lambda i: (0, 0)),
        pl.BlockSpec((N_PAD, D), lambda i: (0, 0)),
        pl.BlockSpec((N_PAD, D), lambda i: (0, 0)),
        pl.BlockSpec((EB, D), lambda i: (i, 0)),
    ],
    out_shape=[
        jax.ShapeDtypeStruct((N_PAD, D), jnp.float32),
        jax.ShapeDtypeStruct((N_PAD, D), jnp.float32),
        jax.ShapeDtypeStruct((N_PAD, D), jnp.float32),
        jax.ShapeDtypeStruct((N_PAD, D), jnp.float32),
        jax.ShapeDtypeStruct((E_PAD, D), jnp.float32),
    ],
)


def _edge_body(q_hbm, k_hbm, v_hbm, ea_hbm, dst_hbm, src_hbm, out_hbm,
               za, vb, dsti, srci, agg_sh, sea, sq, sk, sv, ssc):
    cid = lax.axis_index("c")
    sid = lax.axis_index("s")
    nch = CH0 + cid * (CH1 - CH0)
    base = cid * NS * CH0 + sid * nch

    def zrow(r, carry):
        for j in range(D // 16):
            za[0, r, pl.ds(j * 16, 16)] = jnp.zeros((16,), jnp.float32)
        return carry

    lax.fori_loop(0, C, zrow, 0)
    for t in range(4):
        pltpu.sync_copy(za.at[0], agg_sh.at[pl.ds(sid * RPT + t * C, C)])
    pltpu.sync_copy(za.at[0, pl.ds(0, RPT - 4 * C)],
                    agg_sh.at[pl.ds(sid * RPT + 4 * C, RPT - 4 * C)])
    plsc.subcore_barrier()

    def ea_slice(c):
        return ea_hbm.at[pl.ds((base + c) * C, C)]

    pltpu.sync_copy(dst_hbm.at[base], dsti.at[0])
    pltpu.sync_copy(src_hbm.at[base], srci.at[0])
    pltpu.sync_copy(ea_slice(0), za.at[0])
    pltpu.async_copy(q_hbm.at[dsti.at[0]], za.at[0], sq, add=True).wait()
    pltpu.async_copy(k_hbm.at[srci.at[0]], za.at[0], sk, add=True)
    pltpu.async_copy(v_hbm.at[srci.at[0]], vb, sv)

    def chunk_step(c, par):
        op = 1 - par
        @pl.when(c >= 1)
        def _():
            pltpu.make_async_copy(za.at[op], agg_sh.at[dsti.at[op]],
                                  ssc).wait()

        has_next = c <= nch - 2

        @pl.when(has_next)
        def _():
            pltpu.sync_copy(dst_hbm.at[base + c + 1], dsti.at[op])
            pltpu.sync_copy(src_hbm.at[base + c + 1], srci.at[op])
            pltpu.async_copy(ea_slice(c + 1), za.at[op], sea)

        pltpu.make_async_copy(k_hbm.at[srci.at[par]], za.at[par], sk).wait()
        pltpu.make_async_copy(v_hbm.at[srci.at[par]], vb, sv).wait()

        def row(r, rcarry):
            for j in range(D // 16):
                sl = pl.ds(j * 16, 16)
                za[par, r, sl] = vb[r, sl] / (1.0 + jnp.exp(-za[par, r, sl]))
            return rcarry

        lax.fori_loop(0, C, row, 0)
        pltpu.async_copy(za.at[par], agg_sh.at[dsti.at[par]], ssc, add=True)

        @pl.when(has_next)
        def _():
            pltpu.make_async_copy(ea_slice(c + 1), za.at[op], sea).wait()
            pltpu.async_copy(q_hbm.at[dsti.at[op]], za.at[op], sq, add=True)
            pltpu.async_copy(v_hbm.at[srci.at[op]], vb, sv)
            pltpu.make_async_copy(q_hbm.at[dsti.at[op]], za.at[op], sq).wait()
            pltpu.async_copy(k_hbm.at[srci.at[op]], za.at[op], sk, add=True)

    def pair(t, carry):
        chunk_step(2 * t, 0)
        chunk_step(2 * t + 1, 1)
        return carry

    lax.fori_loop(0, nch // 2, pair, 0)
    pltpu.make_async_copy(za.at[1], agg_sh.at[dsti.at[1]], ssc).wait()
    plsc.subcore_barrier()

    for t in range(4):
        off = sid * RPT + t * C
        pltpu.sync_copy(agg_sh.at[pl.ds(off, C)], za.at[0])
        pltpu.sync_copy(za.at[0], out_hbm.at[cid, pl.ds(off, C)])
    tail = RPT - 4 * C
    off = sid * RPT + 4 * C
    pltpu.sync_copy(agg_sh.at[pl.ds(off, tail)], za.at[0, pl.ds(0, tail)])
    pltpu.sync_copy(za.at[0, pl.ds(0, tail)], out_hbm.at[cid, pl.ds(off, tail)])


_edge = pl.kernel(
    _edge_body,
    out_type=jax.ShapeDtypeStruct((NC, N_PAD, D), jnp.float32),
    mesh=plsc.VectorSubcoreMesh(core_axis_name="c", subcore_axis_name="s"),
    scratch_types=[
        pltpu.VMEM((2, C, D), jnp.float32),
        pltpu.VMEM((C, D), jnp.float32),
        pltpu.VMEM((2, C), jnp.int32),
        pltpu.VMEM((2, C), jnp.int32),
        pltpu.VMEM_SHARED((N_PAD, D), jnp.float32),
        pltpu.SemaphoreType.DMA,
        pltpu.SemaphoreType.DMA,
        pltpu.SemaphoreType.DMA,
        pltpu.SemaphoreType.DMA,
        pltpu.SemaphoreType.DMA,
    ],
)


def _pool_body(xr_ref, a0_ref, a1_ref, batch_ref, out_ref):
    out = jnp.maximum(xr_ref[...] + a0_ref[...] + a1_ref[...], 0.0)
    sel = (batch_ref[...] ==
           lax.broadcasted_iota(jnp.int32, (N, G), 1)).astype(jnp.float32)
    psum = lax.dot_general(sel, out, (((0,), (0,)), ((), ())),
                           preferred_element_type=jnp.float32)
    cnts = lax.dot_general(sel, jnp.ones((N, D), jnp.float32),
                           (((0,), (0,)), ((), ())),
                           preferred_element_type=jnp.float32)
    out_ref[...] = psum / jnp.maximum(cnts, 1.0)


_pool = pl.pallas_call(
    _pool_body,
    out_shape=jax.ShapeDtypeStruct((G, D), jnp.float32),
)


def kernel(x, edge_index, edge_attr, batch, Wq, Wk, Wv, We, Wroot, b):
    pad_e = E_PAD - E
    src3 = jnp.concatenate(
        [edge_index[0], jnp.full((pad_e,), ZROW, jnp.int32)]
    ).reshape(GCHUNK, C)
    dst3 = jnp.concatenate(
        [edge_index[1], jnp.zeros((pad_e,), jnp.int32)]
    ).reshape(GCHUNK, C)
    x_pad = jnp.pad(x, ((0, N_PAD - N), (0, 0)))
    ea_pad = jnp.pad(edge_attr, ((0, pad_e), (0, 0)))
    q, k, v, xr, ea = _prep(x_pad, Wq, Wk, Wv, Wroot, We, b.reshape(1, D),
                            ea_pad)
    aggs = _edge(q, k, v, ea, dst3, src3)
    return _pool(xr[:N], aggs[0, :N], aggs[1, :N], batch.reshape(N, 1))
